# baseline (speedup 1.0000x reference)
import functools

import jax
import jax.numpy as jnp
from jax import lax
from jax.experimental import pallas as pl
from jax.experimental.pallas import tpu as pltpu
from jax.experimental.pallas import tpu_sc as plsc

_NC, _NS = 2, 16
_NW = _NC * _NS
_CH = 1024
_F = _CH // 4
_JB = _F // 128


def _gather16(vec, idx):
    return vec.at[idx].get(mode="promise_in_bounds")


def _sc_body(n_chunks, x_ref, emb_ref, out_ref, t4_ref,
             embbuf, idxbuf, cibuf, outbuf, isem, gsem, wsem):
    wid = lax.axis_index("s") * _NC + lax.axis_index("c")
    lanes = lax.iota(jnp.int32, 16)
    colmod = lanes & 3
    lgrp = lanes >> 2

    pltpu.sync_copy(emb_ref, embbuf)
    e = embbuf[...]

    def build_row(j, carry):
        for u in range(8):
            rsel = (j >> (2 * (u // 2))) & 3
            outbuf[0, j, pl.ds(16 * u, 16)] = _gather16(e, rsel * 4 + colmod)
        return carry

    lax.fori_loop(0, 256, build_row, 0)
    pltpu.sync_copy(outbuf.at[0], t4_ref)

    def xslice(c):
        return x_ref.at[pl.ds((wid * n_chunks + c) * (_CH // 128), _CH // 128)]

    def oslice(c):
        return out_ref.at[pl.ds((wid * n_chunks + c) * _F, _F)]

    def chunk(c, carry):
        buf = c & 1

        @pl.when(c >= 2)
        def _():
            pltpu.make_async_copy(outbuf.at[buf], oslice(c - 2), wsem).wait()

        @pl.when(c == 0)
        def _():
            pltpu.async_copy(xslice(0), idxbuf.at[0], isem)

        pltpu.make_async_copy(xslice(c), idxbuf.at[buf], isem).wait()

        @pl.when(c + 1 < n_chunks)
        def _():
            pltpu.async_copy(xslice(c + 1), idxbuf.at[1 - buf], isem)

        for g in range(16):
            a = []
            for m in range(4):
                f = 64 * g + 16 * m
                a.append(idxbuf[buf, f // 128, pl.ds(f % 128, 16)])
            ci = jnp.zeros((16,), jnp.int32)
            for i in range(4):
                q = [_gather16(a[m], 4 * colmod + i) for m in range(4)]
                lo = jnp.where(lgrp == 0, q[0], q[1])
                hi = jnp.where(lgrp == 2, q[2], q[3])
                ci = ci + (jnp.where(lgrp < 2, lo, hi) << (2 * i))
            cibuf[g // 8, pl.ds((g % 8) * 16, 16)] = ci

        gathers = [
            pltpu.async_copy(
                t4_ref.at[cibuf.at[j]],
                outbuf.at[buf, pl.ds(j * 128, 128)],
                gsem,
            )
            for j in range(_JB)
        ]
        for gth in gathers:
            gth.wait()

        pltpu.async_copy(outbuf.at[buf], oslice(c), wsem)
        return carry

    lax.fori_loop(0, n_chunks, chunk, 0)

    for c in (n_chunks - 2, n_chunks - 1):
        pltpu.make_async_copy(outbuf.at[c & 1], oslice(c), wsem).wait()


def kernel(x, emb_weight):
    B, L = x.shape
    N = B * L
    n_chunks = N // (_NW * _CH)
    x2d = x.reshape(N // 128, 128)
    emb_flat = emb_weight.reshape(16)
    out2d, _ = pl.kernel(
        functools.partial(_sc_body, n_chunks),
        out_type=[
            jax.ShapeDtypeStruct((N // 4, 128), jnp.float32),
            jax.ShapeDtypeStruct((256, 128), jnp.float32),
        ],
        mesh=plsc.VectorSubcoreMesh(core_axis_name="c", subcore_axis_name="s"),
        scratch_types=[
            pltpu.VMEM((16,), jnp.float32),
            pltpu.VMEM((2, _CH // 128, 128), jnp.int32),
            pltpu.VMEM((_JB, 128), jnp.int32),
            pltpu.VMEM((2, _F, 128), jnp.float32),
            pltpu.SemaphoreType.DMA,
            pltpu.SemaphoreType.DMA,
            pltpu.SemaphoreType.DMA,
        ],
    )(x2d, emb_flat)
    return out2d.reshape(B, L, 32)


# speedup vs baseline: 11.5514x; 11.5514x over previous
import jax
import jax.numpy as jnp
from jax.experimental import pallas as pl

_LB = 8
_BB = 2048


def _body(xt_ref, emb_ref, o_ref):
    xv = xt_ref[...]
    emb = emb_ref[...]
    t = jnp.concatenate([emb] * 8, axis=1)
    x3 = xv[:, None, :]
    r0 = t[0][None, :, None]
    r1 = t[1][None, :, None]
    r2 = t[2][None, :, None]
    r3 = t[3][None, :, None]
    lo = jnp.where(x3 == 0, r0, r1)
    hi = jnp.where(x3 == 2, r2, r3)
    o_ref[...] = jnp.where(x3 < 2, lo, hi)


def kernel(x, emb_weight):
    B, L = x.shape
    xT = x.T
    grid = (L // _LB, B // _BB)
    out = pl.pallas_call(
        _body,
        grid=grid,
        in_specs=[
            pl.BlockSpec((_LB, _BB), lambda i, j: (i, j)),
            pl.BlockSpec((4, 4), lambda i, j: (0, 0)),
        ],
        out_specs=pl.BlockSpec((_LB, 32, _BB), lambda i, j: (i, 0, j)),
        out_shape=jax.ShapeDtypeStruct((L, 32, B), jnp.float32),
    )(xT, emb_weight)
    return jnp.transpose(out, (2, 0, 1))


# speedup vs baseline: 16.8156x; 1.4557x over previous
import jax
import jax.numpy as jnp
from jax.experimental import pallas as pl

_LB = 8
_BB = 4096


def _body(xt_ref, emb_ref, o_ref):
    xv = xt_ref[...]
    emb = emb_ref[...]
    xf = xv.astype(jnp.float32)
    e0, e1, e2, e3 = emb[0], emb[1], emb[2], emb[3]
    p0 = e0
    p1 = (-11.0 * e0 + 18.0 * e1 - 9.0 * e2 + 2.0 * e3) / 6.0
    p2 = (2.0 * e0 - 5.0 * e1 + 4.0 * e2 - e3) / 2.0
    p3 = (-e0 + 3.0 * e1 - 3.0 * e2 + e3) / 6.0
    for cc in range(4):
        v = ((p3[cc] * xf + p2[cc]) * xf + p1[cc]) * xf + p0[cc]
        for k in range(8):
            o_ref[:, 4 * k + cc, :] = v


def kernel(x, emb_weight):
    B, L = x.shape
    xT = x.T
    grid = (L // _LB, B // _BB)
    out = pl.pallas_call(
        _body,
        grid=grid,
        in_specs=[
            pl.BlockSpec((_LB, _BB), lambda i, j: (i, j)),
            pl.BlockSpec((4, 4), lambda i, j: (0, 0)),
        ],
        out_specs=pl.BlockSpec((_LB, 32, _BB), lambda i, j: (i, 0, j)),
        out_shape=jax.ShapeDtypeStruct((L, 32, B), jnp.float32),
    )(xT, emb_weight)
    return jnp.transpose(out, (2, 0, 1))


# speedup vs baseline: 19.4364x; 1.1559x over previous
import jax
import jax.numpy as jnp
from jax.experimental import pallas as pl

_LB = 8
_BB = 8192


def _body(xt_ref, emb_ref, o_ref):
    xv = xt_ref[...]
    emb = emb_ref[...]
    xf = xv.astype(jnp.float32)
    e0, e1, e2, e3 = emb[0], emb[1], emb[2], emb[3]
    p0 = e0
    p1 = (-11.0 * e0 + 18.0 * e1 - 9.0 * e2 + 2.0 * e3) / 6.0
    p2 = (2.0 * e0 - 5.0 * e1 + 4.0 * e2 - e3) / 2.0
    p3 = (-e0 + 3.0 * e1 - 3.0 * e2 + e3) / 6.0
    for cc in range(4):
        v = ((p3[cc] * xf + p2[cc]) * xf + p1[cc]) * xf + p0[cc]
        for k in range(8):
            o_ref[:, 4 * k + cc, :] = v


def kernel(x, emb_weight):
    B, L = x.shape
    xT = x.T
    grid = (L // _LB, B // _BB)
    out = pl.pallas_call(
        _body,
        grid=grid,
        in_specs=[
            pl.BlockSpec((_LB, _BB), lambda i, j: (i, j)),
            pl.BlockSpec((4, 4), lambda i, j: (0, 0)),
        ],
        out_specs=pl.BlockSpec((_LB, 32, _BB), lambda i, j: (i, 0, j)),
        out_shape=jax.ShapeDtypeStruct((L, 32, B), jnp.float32),
    )(xT, emb_weight)
    return jnp.transpose(out, (2, 0, 1))
